# baseline (speedup 1.0000x reference)
import functools

import jax
import jax.numpy as jnp
from jax import lax
from jax.experimental import pallas as pl
from jax.experimental.pallas import tpu as pltpu
from jax.experimental.pallas import tpu_sc as plsc

V = 32000
D = 1024
H = 2048
E = 8
K = 2
L = 2
T = 2048


def _emb_gather_sc(ids, emb):
    info = plsc.get_sparse_core_info()
    nw = info.num_cores * info.num_subcores
    b_per_w = T // nw
    mesh = plsc.VectorSubcoreMesh(core_axis_name="c", subcore_axis_name="s")

    @functools.partial(
        pl.kernel,
        mesh=mesh,
        out_type=jax.ShapeDtypeStruct((T, D), jnp.float32),
        scratch_types=[
            pltpu.VMEM((b_per_w,), jnp.int32),
            pltpu.VMEM((b_per_w, D), jnp.float32),
            pltpu.SemaphoreType.DMA,
        ],
    )
    def emb_k(ids_hbm, emb_hbm, out_hbm, idx_v, rows_v, sem):
        wid = lax.axis_index("s") * info.num_cores + lax.axis_index("c")
        base = wid * b_per_w
        pltpu.sync_copy(ids_hbm.at[pl.ds(base, b_per_w)], idx_v)
        pltpu.async_copy(emb_hbm.at[idx_v], rows_v, sem).wait()
        pltpu.sync_copy(rows_v, out_hbm.at[pl.ds(base, b_per_w)])

    return emb_k(ids, emb)


_emb_gather = _emb_gather_sc


_RM_TM = 256


def _router_mem_body(x_ref, wr_ref, wq_ref, wk_ref, wv_ref, wb_ref,
                     gate_ref, q_ref, delta_ref, aux_ref, acc, fp_acc):
    i = pl.program_id(0)
    x = x_ref[...]

    logits = jnp.dot(x, wr_ref[...], preferred_element_type=jnp.float32)
    m = jnp.max(logits, axis=-1, keepdims=True)
    ex = jnp.exp(logits - m)
    probs = ex / jnp.sum(ex, axis=-1, keepdims=True)

    idxs = lax.broadcasted_iota(jnp.int32, probs.shape, 1)
    m1 = jnp.max(probs, axis=-1, keepdims=True)
    i1 = jnp.min(jnp.where(probs == m1, idxs, E), axis=-1, keepdims=True)
    oh1 = idxs == i1
    masked = jnp.where(oh1, -jnp.inf, probs)
    m2 = jnp.max(masked, axis=-1, keepdims=True)
    i2 = jnp.min(jnp.where(masked == m2, idxs, E), axis=-1, keepdims=True)
    oh2 = idxs == i2
    s = m1 + m2
    gate_ref[...] = (jnp.where(oh1, m1 / s, 0.0)
                     + jnp.where(oh2, m2 / s, 0.0))

    f_part = jnp.sum((oh1 | oh2).astype(jnp.float32), axis=0, keepdims=True)
    p_part = jnp.sum(probs, axis=0, keepdims=True)
    fp_part = jnp.concatenate([f_part, p_part], axis=0)

    q_ref[...] = jnp.dot(x, wq_ref[...], preferred_element_type=jnp.float32)
    k = jnp.dot(x, wk_ref[...], preferred_element_type=jnp.float32)
    v = jnp.dot(x, wv_ref[...], preferred_element_type=jnp.float32)
    beta = jax.nn.sigmoid(
        jnp.dot(x, wb_ref[...], preferred_element_type=jnp.float32))
    part = lax.dot_general(k * beta, v, (((0,), (0,)), ((), ())),
                           preferred_element_type=jnp.float32)

    @pl.when(i == 0)
    def _():
        acc[...] = part
        fp_acc[...] = fp_part

    @pl.when(i != 0)
    def _():
        acc[...] += part
        fp_acc[...] += fp_part

    @pl.when(i == pl.num_programs(0) - 1)
    def _():
        delta_ref[...] = acc[...] * (1.0 / T)
        fp = fp_acc[...] * (1.0 / T)
        aux_ref[0, 0] = E * jnp.sum(fp[0] * fp[1])


def _router_mem(x, wr, wq, wk, wv, wb):
    grid = (T // _RM_TM,)
    return pl.pallas_call(
        _router_mem_body,
        grid=grid,
        out_shape=(
            jax.ShapeDtypeStruct((T, E), jnp.float32),
            jax.ShapeDtypeStruct((T, D), jnp.float32),
            jax.ShapeDtypeStruct((D, D), jnp.float32),
            jax.ShapeDtypeStruct((1, 1), jnp.float32),
        ),
        in_specs=[
            pl.BlockSpec((_RM_TM, D), lambda i: (i, 0)),
            pl.BlockSpec((D, E), lambda i: (0, 0)),
            pl.BlockSpec((D, D), lambda i: (0, 0)),
            pl.BlockSpec((D, D), lambda i: (0, 0)),
            pl.BlockSpec((D, D), lambda i: (0, 0)),
            pl.BlockSpec((D, 1), lambda i: (0, 0)),
        ],
        out_specs=(
            pl.BlockSpec((_RM_TM, E), lambda i: (i, 0)),
            pl.BlockSpec((_RM_TM, D), lambda i: (i, 0)),
            pl.BlockSpec((D, D), lambda i: (0, 0)),
            pl.BlockSpec(memory_space=pltpu.SMEM),
        ),
        scratch_shapes=[
            pltpu.VMEM((D, D), jnp.float32),
            pltpu.VMEM((2, E), jnp.float32),
        ],
    )(x, wr, wq, wk, wv, wb)


_FFN_TM = 256


def _ffn_out_body(x_ref, w1_ref, b1_ref, w2_ref, b2_ref, gate_ref, q_ref,
                  delta_ref, out_ref, acc):
    e = pl.program_id(0)
    i = pl.program_id(1)
    x = x_ref[...]
    h1 = jnp.maximum(
        jnp.dot(x, w1_ref[0], preferred_element_type=jnp.float32)
        + b1_ref[0], 0.0)
    y = jnp.dot(h1, w2_ref[0], preferred_element_type=jnp.float32) + b2_ref[0]
    lanes = lax.broadcasted_iota(jnp.int32, (gate_ref.shape[0], E), 1)
    g = jnp.sum(jnp.where(lanes == e, gate_ref[...], 0.0), axis=1,
                keepdims=True)
    contrib = g * y

    @pl.when(e == 0)
    def _():
        acc[i] = contrib

    @pl.when((e != 0) & (e != E - 1))
    def _():
        acc[i] += contrib

    @pl.when(e == E - 1)
    def _():
        read = jnp.dot(q_ref[...], delta_ref[...],
                       preferred_element_type=jnp.float32)
        out_ref[...] = x + acc[i] + contrib + read


def _ffn_out(x, w1, b1, w2, b2, gate, q, delta):
    nt = T // _FFN_TM
    grid = (E, nt)
    return pl.pallas_call(
        _ffn_out_body,
        grid=grid,
        out_shape=jax.ShapeDtypeStruct((T, D), jnp.float32),
        in_specs=[
            pl.BlockSpec((_FFN_TM, D), lambda e, i: (i, 0)),
            pl.BlockSpec((1, D, H), lambda e, i: (e, 0, 0)),
            pl.BlockSpec((1, 1, H), lambda e, i: (e, 0, 0)),
            pl.BlockSpec((1, H, D), lambda e, i: (e, 0, 0)),
            pl.BlockSpec((1, 1, D), lambda e, i: (e, 0, 0)),
            pl.BlockSpec((_FFN_TM, E), lambda e, i: (i, 0)),
            pl.BlockSpec((_FFN_TM, D), lambda e, i: (i, 0)),
            pl.BlockSpec((D, D), lambda e, i: (0, 0)),
        ],
        out_specs=pl.BlockSpec(
            (_FFN_TM, D), lambda e, i: (jnp.where(e == E - 1, i, 0), 0)),
        scratch_shapes=[pltpu.VMEM((nt, _FFN_TM, D), jnp.float32)],
    )(x, w1, b1.reshape(E, 1, H), w2, b2.reshape(E, 1, D), gate, q, delta)


_HEAD_TM = 512
_HEAD_VN = 3200


def _head_body(x_ref, g_ref, b_ref, hw_ref, out_ref):
    x = x_ref[...]
    mu = jnp.mean(x, axis=-1, keepdims=True)
    xc = x - mu
    var = jnp.mean(xc * xc, axis=-1, keepdims=True)
    xn = xc * lax.rsqrt(var + 1e-5) * g_ref[...] + b_ref[...]
    out_ref[...] = lax.dot_general(xn, hw_ref[...],
                                   (((1,), (1,)), ((), ())),
                                   preferred_element_type=jnp.float32)


def _head(x, ln_g, ln_b, head_w):
    grid = (V // _HEAD_VN, T // _HEAD_TM)
    return pl.pallas_call(
        _head_body,
        grid=grid,
        out_shape=jax.ShapeDtypeStruct((T, V), jnp.float32),
        in_specs=[
            pl.BlockSpec((_HEAD_TM, D), lambda j, i: (i, 0)),
            pl.BlockSpec((1, D), lambda j, i: (0, 0)),
            pl.BlockSpec((1, D), lambda j, i: (0, 0)),
            pl.BlockSpec((_HEAD_VN, D), lambda j, i: (j, 0)),
        ],
        out_specs=pl.BlockSpec((_HEAD_TM, _HEAD_VN), lambda j, i: (i, j)),
    )(x, ln_g, ln_b, head_w)


def kernel(input_ids, emb, Wr, W1, b1, W2, b2, Wq, Wk, Wv, Wb, ln_g, ln_b, head_w):
    ids = input_ids.reshape(T).astype(jnp.int32)
    x = _emb_gather(ids, emb)

    aux_total = jnp.zeros((), jnp.float32)
    for l in range(L):
        gate, q, delta, aux = _router_mem(x, Wr[l], Wq[l], Wk[l], Wv[l], Wb[l])
        x = _ffn_out(x, W1[l], b1[l], W2[l], b2[l], gate, q, delta)
        aux_total = aux_total + aux[0, 0]

    logits = _head(x, ln_g.reshape(1, D), ln_b.reshape(1, D), head_w)
    return logits.reshape(1, T, V), aux_total


# speedup vs baseline: 1.0551x; 1.0551x over previous
import functools

import jax
import jax.numpy as jnp
from jax import lax
from jax.experimental import pallas as pl
from jax.experimental.pallas import tpu as pltpu
from jax.experimental.pallas import tpu_sc as plsc

V = 32000
D = 1024
H = 2048
E = 8
K = 2
L = 2
T = 2048


def _emb_gather_sc(ids, emb):
    info = plsc.get_sparse_core_info()
    nw = info.num_cores * info.num_subcores
    b_per_w = T // nw
    mesh = plsc.VectorSubcoreMesh(core_axis_name="c", subcore_axis_name="s")

    @functools.partial(
        pl.kernel,
        mesh=mesh,
        out_type=jax.ShapeDtypeStruct((T, D), jnp.float32),
        scratch_types=[
            pltpu.VMEM((b_per_w,), jnp.int32),
            pltpu.VMEM((b_per_w, D), jnp.float32),
            pltpu.SemaphoreType.DMA,
        ],
    )
    def emb_k(ids_hbm, emb_hbm, out_hbm, idx_v, rows_v, sem):
        wid = lax.axis_index("s") * info.num_cores + lax.axis_index("c")
        base = wid * b_per_w
        pltpu.sync_copy(ids_hbm.at[pl.ds(base, b_per_w)], idx_v)
        pltpu.async_copy(emb_hbm.at[idx_v], rows_v, sem).wait()
        pltpu.sync_copy(rows_v, out_hbm.at[pl.ds(base, b_per_w)])

    return emb_k(ids, emb)


_emb_gather = _emb_gather_sc


_RM_TM = 256


def _router_mem_body(x_ref, wr_ref, wq_ref, wk_ref, wv_ref, wb_ref,
                     gate_ref, q_ref, delta_ref, aux_ref, acc, fp_acc):
    i = pl.program_id(0)
    x = x_ref[...]

    logits = jnp.dot(x, wr_ref[...], preferred_element_type=jnp.float32)
    m = jnp.max(logits, axis=-1, keepdims=True)
    ex = jnp.exp(logits - m)
    probs = ex / jnp.sum(ex, axis=-1, keepdims=True)

    idxs = lax.broadcasted_iota(jnp.int32, probs.shape, 1)
    m1 = jnp.max(probs, axis=-1, keepdims=True)
    i1 = jnp.min(jnp.where(probs == m1, idxs, E), axis=-1, keepdims=True)
    oh1 = idxs == i1
    masked = jnp.where(oh1, -jnp.inf, probs)
    m2 = jnp.max(masked, axis=-1, keepdims=True)
    i2 = jnp.min(jnp.where(masked == m2, idxs, E), axis=-1, keepdims=True)
    oh2 = idxs == i2
    s = m1 + m2
    gate_ref[...] = (jnp.where(oh1, m1 / s, 0.0)
                     + jnp.where(oh2, m2 / s, 0.0))

    f_part = jnp.sum((oh1 | oh2).astype(jnp.float32), axis=0, keepdims=True)
    p_part = jnp.sum(probs, axis=0, keepdims=True)
    fp_part = jnp.concatenate([f_part, p_part], axis=0)

    q_ref[...] = jnp.dot(x, wq_ref[...], preferred_element_type=jnp.float32)
    k = jnp.dot(x, wk_ref[...], preferred_element_type=jnp.float32)
    v = jnp.dot(x, wv_ref[...], preferred_element_type=jnp.float32)
    beta = jax.nn.sigmoid(
        jnp.dot(x, wb_ref[...], preferred_element_type=jnp.float32))
    part = lax.dot_general(k * beta, v, (((0,), (0,)), ((), ())),
                           preferred_element_type=jnp.float32)

    @pl.when(i == 0)
    def _():
        acc[...] = part
        fp_acc[...] = fp_part

    @pl.when(i != 0)
    def _():
        acc[...] += part
        fp_acc[...] += fp_part

    @pl.when(i == pl.num_programs(0) - 1)
    def _():
        delta_ref[...] = acc[...] * (1.0 / T)
        fp = fp_acc[...] * (1.0 / T)
        aux_ref[0, 0] = E * jnp.sum(fp[0] * fp[1])


def _router_mem(x, wr, wq, wk, wv, wb):
    grid = (T // _RM_TM,)
    return pl.pallas_call(
        _router_mem_body,
        grid=grid,
        out_shape=(
            jax.ShapeDtypeStruct((T, E), jnp.float32),
            jax.ShapeDtypeStruct((T, D), jnp.float32),
            jax.ShapeDtypeStruct((D, D), jnp.float32),
            jax.ShapeDtypeStruct((1, 1), jnp.float32),
        ),
        in_specs=[
            pl.BlockSpec((_RM_TM, D), lambda i: (i, 0)),
            pl.BlockSpec((D, E), lambda i: (0, 0)),
            pl.BlockSpec((D, D), lambda i: (0, 0)),
            pl.BlockSpec((D, D), lambda i: (0, 0)),
            pl.BlockSpec((D, D), lambda i: (0, 0)),
            pl.BlockSpec((D, 1), lambda i: (0, 0)),
        ],
        out_specs=(
            pl.BlockSpec((_RM_TM, E), lambda i: (i, 0)),
            pl.BlockSpec((_RM_TM, D), lambda i: (i, 0)),
            pl.BlockSpec((D, D), lambda i: (0, 0)),
            pl.BlockSpec(memory_space=pltpu.SMEM),
        ),
        scratch_shapes=[
            pltpu.VMEM((D, D), jnp.float32),
            pltpu.VMEM((2, E), jnp.float32),
        ],
    )(x, wr, wq, wk, wv, wb)


_FFN_TM = 512


def _ffn_out_body(x_ref, w1_ref, b1_ref, w2_ref, b2_ref, gate_ref,
                  out_ref, acc):
    e = pl.program_id(0)
    i = pl.program_id(1)
    h1 = jnp.maximum(
        jnp.dot(x_ref[...], w1_ref[0], preferred_element_type=jnp.float32)
        + b1_ref[0], 0.0)
    y = jnp.dot(h1, w2_ref[0], preferred_element_type=jnp.float32) + b2_ref[0]
    lanes = lax.broadcasted_iota(jnp.int32, (gate_ref.shape[0], E), 1)
    g = jnp.sum(jnp.where(lanes == e, gate_ref[...], 0.0), axis=1,
                keepdims=True)
    contrib = g * y

    @pl.when(e == 0)
    def _():
        acc[i] = contrib

    @pl.when((e != 0) & (e != E - 1))
    def _():
        acc[i] += contrib

    @pl.when(e == E - 1)
    def _():
        out_ref[...] = x_ref[...] + acc[i] + contrib


def _ffn_out(x, w1, b1, w2, b2, gate):
    nt = T // _FFN_TM
    grid = (E, nt)
    return pl.pallas_call(
        _ffn_out_body,
        grid=grid,
        out_shape=jax.ShapeDtypeStruct((T, D), jnp.float32),
        in_specs=[
            pl.BlockSpec((_FFN_TM, D), lambda e, i: (i, 0)),
            pl.BlockSpec((1, D, H), lambda e, i: (e, 0, 0)),
            pl.BlockSpec((1, 1, H), lambda e, i: (e, 0, 0)),
            pl.BlockSpec((1, H, D), lambda e, i: (e, 0, 0)),
            pl.BlockSpec((1, 1, D), lambda e, i: (e, 0, 0)),
            pl.BlockSpec((_FFN_TM, E), lambda e, i: (i, 0)),
        ],
        out_specs=pl.BlockSpec(
            (_FFN_TM, D), lambda e, i: (jnp.where(e == E - 1, i, 0), 0)),
        scratch_shapes=[pltpu.VMEM((nt, _FFN_TM, D), jnp.float32)],
    )(x, w1, b1.reshape(E, 1, H), w2, b2.reshape(E, 1, D), gate)


_OUT_TM = 512


def _read_body(xf_ref, q_ref, delta_ref, out_ref):
    read = jnp.dot(q_ref[...], delta_ref[...],
                   preferred_element_type=jnp.float32)
    out_ref[...] = xf_ref[...] + read


def _read_combine(xf, q, delta):
    grid = (T // _OUT_TM,)
    return pl.pallas_call(
        _read_body,
        grid=grid,
        out_shape=jax.ShapeDtypeStruct((T, D), jnp.float32),
        in_specs=[
            pl.BlockSpec((_OUT_TM, D), lambda i: (i, 0)),
            pl.BlockSpec((_OUT_TM, D), lambda i: (i, 0)),
            pl.BlockSpec((D, D), lambda i: (0, 0)),
        ],
        out_specs=pl.BlockSpec((_OUT_TM, D), lambda i: (i, 0)),
    )(xf, q, delta)


_HEAD_TM = 512
_HEAD_VN = 3200


def _head_body(x_ref, g_ref, b_ref, hw_ref, out_ref):
    x = x_ref[...]
    mu = jnp.mean(x, axis=-1, keepdims=True)
    xc = x - mu
    var = jnp.mean(xc * xc, axis=-1, keepdims=True)
    xn = xc * lax.rsqrt(var + 1e-5) * g_ref[...] + b_ref[...]
    out_ref[...] = lax.dot_general(xn, hw_ref[...],
                                   (((1,), (1,)), ((), ())),
                                   preferred_element_type=jnp.float32)


def _head(x, ln_g, ln_b, head_w):
    grid = (V // _HEAD_VN, T // _HEAD_TM)
    return pl.pallas_call(
        _head_body,
        grid=grid,
        out_shape=jax.ShapeDtypeStruct((T, V), jnp.float32),
        in_specs=[
            pl.BlockSpec((_HEAD_TM, D), lambda j, i: (i, 0)),
            pl.BlockSpec((1, D), lambda j, i: (0, 0)),
            pl.BlockSpec((1, D), lambda j, i: (0, 0)),
            pl.BlockSpec((_HEAD_VN, D), lambda j, i: (j, 0)),
        ],
        out_specs=pl.BlockSpec((_HEAD_TM, _HEAD_VN), lambda j, i: (i, j)),
    )(x, ln_g, ln_b, head_w)


def kernel(input_ids, emb, Wr, W1, b1, W2, b2, Wq, Wk, Wv, Wb, ln_g, ln_b, head_w):
    ids = input_ids.reshape(T).astype(jnp.int32)
    x = _emb_gather(ids, emb)

    aux_total = jnp.zeros((), jnp.float32)
    for l in range(L):
        gate, q, delta, aux = _router_mem(x, Wr[l], Wq[l], Wk[l], Wv[l], Wb[l])
        xf = _ffn_out(x, W1[l], b1[l], W2[l], b2[l], gate)
        x = _read_combine(xf, q, delta)
        aux_total = aux_total + aux[0, 0]

    logits = _head(x, ln_g.reshape(1, D), ln_b.reshape(1, D), head_w)
    return logits.reshape(1, T, V), aux_total


# speedup vs baseline: 1.0668x; 1.0111x over previous
import functools

import jax
import jax.numpy as jnp
from jax import lax
from jax.experimental import pallas as pl
from jax.experimental.pallas import tpu as pltpu
from jax.experimental.pallas import tpu_sc as plsc

V = 32000
D = 1024
H = 2048
E = 8
K = 2
L = 2
T = 2048


def _emb_gather_sc(ids, emb):
    info = plsc.get_sparse_core_info()
    nw = info.num_cores * info.num_subcores
    b_per_w = T // nw
    mesh = plsc.VectorSubcoreMesh(core_axis_name="c", subcore_axis_name="s")

    @functools.partial(
        pl.kernel,
        mesh=mesh,
        out_type=jax.ShapeDtypeStruct((T, D), jnp.float32),
        scratch_types=[
            pltpu.VMEM((b_per_w,), jnp.int32),
            pltpu.VMEM((b_per_w, D), jnp.float32),
            pltpu.SemaphoreType.DMA,
        ],
    )
    def emb_k(ids_hbm, emb_hbm, out_hbm, idx_v, rows_v, sem):
        wid = lax.axis_index("s") * info.num_cores + lax.axis_index("c")
        base = wid * b_per_w
        pltpu.sync_copy(ids_hbm.at[pl.ds(base, b_per_w)], idx_v)
        pltpu.async_copy(emb_hbm.at[idx_v], rows_v, sem).wait()
        pltpu.sync_copy(rows_v, out_hbm.at[pl.ds(base, b_per_w)])

    return emb_k(ids, emb)


_emb_gather = _emb_gather_sc


_RM_TM = 512


def _router_mem_body(x_ref, wr_ref, wq_ref, wk_ref, wv_ref, wb_ref,
                     gate_ref, q_ref, delta_ref, aux_ref, acc, fp_acc):
    i = pl.program_id(0)
    x = x_ref[...]

    logits = jnp.dot(x, wr_ref[...], preferred_element_type=jnp.float32)
    m = jnp.max(logits, axis=-1, keepdims=True)
    ex = jnp.exp(logits - m)
    probs = ex / jnp.sum(ex, axis=-1, keepdims=True)

    idxs = lax.broadcasted_iota(jnp.int32, probs.shape, 1)
    m1 = jnp.max(probs, axis=-1, keepdims=True)
    i1 = jnp.min(jnp.where(probs == m1, idxs, E), axis=-1, keepdims=True)
    oh1 = idxs == i1
    masked = jnp.where(oh1, -jnp.inf, probs)
    m2 = jnp.max(masked, axis=-1, keepdims=True)
    i2 = jnp.min(jnp.where(masked == m2, idxs, E), axis=-1, keepdims=True)
    oh2 = idxs == i2
    s = m1 + m2
    gate_ref[...] = (jnp.where(oh1, m1 / s, 0.0)
                     + jnp.where(oh2, m2 / s, 0.0))

    f_part = jnp.sum((oh1 | oh2).astype(jnp.float32), axis=0, keepdims=True)
    p_part = jnp.sum(probs, axis=0, keepdims=True)
    fp_part = jnp.concatenate([f_part, p_part], axis=0)

    q_ref[...] = jnp.dot(x, wq_ref[...], preferred_element_type=jnp.float32)
    k = jnp.dot(x, wk_ref[...], preferred_element_type=jnp.float32)
    v = jnp.dot(x, wv_ref[...], preferred_element_type=jnp.float32)
    beta = jax.nn.sigmoid(
        jnp.dot(x, wb_ref[...], preferred_element_type=jnp.float32))
    part = lax.dot_general(k * beta, v, (((0,), (0,)), ((), ())),
                           preferred_element_type=jnp.float32)

    @pl.when(i == 0)
    def _():
        acc[...] = part
        fp_acc[...] = fp_part

    @pl.when(i != 0)
    def _():
        acc[...] += part
        fp_acc[...] += fp_part

    @pl.when(i == pl.num_programs(0) - 1)
    def _():
        delta_ref[...] = acc[...] * (1.0 / T)
        fp = fp_acc[...] * (1.0 / T)
        aux_ref[0, 0] = E * jnp.sum(fp[0] * fp[1])


def _router_mem(x, wr, wq, wk, wv, wb):
    grid = (T // _RM_TM,)
    return pl.pallas_call(
        _router_mem_body,
        grid=grid,
        out_shape=(
            jax.ShapeDtypeStruct((T, E), jnp.float32),
            jax.ShapeDtypeStruct((T, D), jnp.float32),
            jax.ShapeDtypeStruct((D, D), jnp.float32),
            jax.ShapeDtypeStruct((1, 1), jnp.float32),
        ),
        in_specs=[
            pl.BlockSpec((_RM_TM, D), lambda i: (i, 0)),
            pl.BlockSpec((D, E), lambda i: (0, 0)),
            pl.BlockSpec((D, D), lambda i: (0, 0)),
            pl.BlockSpec((D, D), lambda i: (0, 0)),
            pl.BlockSpec((D, D), lambda i: (0, 0)),
            pl.BlockSpec((D, 1), lambda i: (0, 0)),
        ],
        out_specs=(
            pl.BlockSpec((_RM_TM, E), lambda i: (i, 0)),
            pl.BlockSpec((_RM_TM, D), lambda i: (i, 0)),
            pl.BlockSpec((D, D), lambda i: (0, 0)),
            pl.BlockSpec(memory_space=pltpu.SMEM),
        ),
        scratch_shapes=[
            pltpu.VMEM((D, D), jnp.float32),
            pltpu.VMEM((2, E), jnp.float32),
        ],
    )(x, wr, wq, wk, wv, wb)


_FFN_TM = 512


def _ffn_out_body(x_ref, w1_ref, b1_ref, w2_ref, b2_ref, gate_ref,
                  out_ref, acc):
    e = pl.program_id(0)
    i = pl.program_id(1)
    h1 = jnp.maximum(
        jnp.dot(x_ref[...], w1_ref[0], preferred_element_type=jnp.float32)
        + b1_ref[0], 0.0)
    y = jnp.dot(h1, w2_ref[0], preferred_element_type=jnp.float32) + b2_ref[0]
    lanes = lax.broadcasted_iota(jnp.int32, (gate_ref.shape[0], E), 1)
    g = jnp.sum(jnp.where(lanes == e, gate_ref[...], 0.0), axis=1,
                keepdims=True)
    contrib = g * y

    @pl.when(e == 0)
    def _():
        acc[i] = contrib

    @pl.when((e != 0) & (e != pl.num_programs(0) - 1))
    def _():
        acc[i] += contrib

    @pl.when(e == pl.num_programs(0) - 1)
    def _():
        out_ref[...] = x_ref[...] + acc[i] + contrib


def _ffn_out(x, w1, b1, w2, b2, gate):
    nt = T // _FFN_TM
    grid = (E, nt)
    return pl.pallas_call(
        _ffn_out_body,
        grid=grid,
        out_shape=jax.ShapeDtypeStruct((T, D), jnp.float32),
        in_specs=[
            pl.BlockSpec((_FFN_TM, D), lambda e, i: (i, 0)),
            pl.BlockSpec((1, D, H), lambda e, i: (e, 0, 0)),
            pl.BlockSpec((1, 1, H), lambda e, i: (e, 0, 0)),
            pl.BlockSpec((1, H, D), lambda e, i: (e, 0, 0)),
            pl.BlockSpec((1, 1, D), lambda e, i: (e, 0, 0)),
            pl.BlockSpec((_FFN_TM, E), lambda e, i: (i, 0)),
        ],
        out_specs=pl.BlockSpec(
            (_FFN_TM, D), lambda e, i: (jnp.where(e == E - 1, i, 0), 0)),
        scratch_shapes=[pltpu.VMEM((nt, _FFN_TM, D), jnp.float32)],
    )(x, w1, b1.reshape(E, 1, H), w2, b2.reshape(E, 1, D), gate)


_OUT_TM = 512


def _read_body(xf_ref, q_ref, delta_ref, out_ref):
    read = jnp.dot(q_ref[...], delta_ref[...],
                   preferred_element_type=jnp.float32)
    out_ref[...] = xf_ref[...] + read


def _read_combine(xf, q, delta):
    grid = (T // _OUT_TM,)
    return pl.pallas_call(
        _read_body,
        grid=grid,
        out_shape=jax.ShapeDtypeStruct((T, D), jnp.float32),
        in_specs=[
            pl.BlockSpec((_OUT_TM, D), lambda i: (i, 0)),
            pl.BlockSpec((_OUT_TM, D), lambda i: (i, 0)),
            pl.BlockSpec((D, D), lambda i: (0, 0)),
        ],
        out_specs=pl.BlockSpec((_OUT_TM, D), lambda i: (i, 0)),
    )(xf, q, delta)


_HEAD_TM = 512
_HEAD_VN = 3200


def _head_body(x_ref, g_ref, b_ref, hw_ref, out_ref):
    x = x_ref[...]
    mu = jnp.mean(x, axis=-1, keepdims=True)
    xc = x - mu
    var = jnp.mean(xc * xc, axis=-1, keepdims=True)
    xn = xc * lax.rsqrt(var + 1e-5) * g_ref[...] + b_ref[...]
    out_ref[...] = lax.dot_general(xn, hw_ref[...],
                                   (((1,), (1,)), ((), ())),
                                   preferred_element_type=jnp.float32)


def _head(x, ln_g, ln_b, head_w):
    grid = (V // _HEAD_VN, T // _HEAD_TM)
    return pl.pallas_call(
        _head_body,
        grid=grid,
        out_shape=jax.ShapeDtypeStruct((T, V), jnp.float32),
        in_specs=[
            pl.BlockSpec((_HEAD_TM, D), lambda j, i: (i, 0)),
            pl.BlockSpec((1, D), lambda j, i: (0, 0)),
            pl.BlockSpec((1, D), lambda j, i: (0, 0)),
            pl.BlockSpec((_HEAD_VN, D), lambda j, i: (j, 0)),
        ],
        out_specs=pl.BlockSpec((_HEAD_TM, _HEAD_VN), lambda j, i: (i, j)),
    )(x, ln_g, ln_b, head_w)


def kernel(input_ids, emb, Wr, W1, b1, W2, b2, Wq, Wk, Wv, Wb, ln_g, ln_b, head_w):
    ids = input_ids.reshape(T).astype(jnp.int32)
    x = _emb_gather(ids, emb)

    aux_total = jnp.zeros((), jnp.float32)
    for l in range(L):
        gate, q, delta, aux = _router_mem(x, Wr[l], Wq[l], Wk[l], Wv[l], Wb[l])
        xf = _ffn_out(x, W1[l], b1[l], W2[l], b2[l], gate)
        x = _read_combine(xf, q, delta)
        aux_total = aux_total + aux[0, 0]

    logits = _head(x, ln_g.reshape(1, D), ln_b.reshape(1, D), head_w)
    return logits.reshape(1, T, V), aux_total
